# Initial kernel scaffold; baseline (speedup 1.0000x reference)
import functools

import jax
import jax.numpy as jnp
from jax import lax
from jax.experimental import pallas as pl
from jax.experimental.pallas import tpu as pltpu
from jax.experimental.pallas import tpu_sc as plsc

C = 128
K = 32
N = 10000

NW = 32
CHUNK = 320
NB = 4
STEPS = CHUNK // NB


def _bn(y, g, b, axis):
    m = jnp.mean(y, axis=axis, keepdims=True)
    v = jnp.mean((y - m) ** 2, axis=axis, keepdims=True)
    return (y - m) * lax.rsqrt(v + 1e-5) * g + b


def _tc1_body(f_ref, w_ref, g_ref, b_ref, o_ref):
    y = lax.dot_general(f_ref[...], w_ref[...], (((0,), (1,)), ((), ())),
                        preferred_element_type=jnp.float32)
    o_ref[...] = jnp.maximum(_bn(y, g_ref[...], b_ref[...], 0), 0.0)


def _tc2_body(fa_ref, f_ref, w2_ref, g2_ref, b2_ref, w3_ref, g3_ref, b3_ref,
              o_ref):
    y2 = lax.dot_general(w2_ref[...], fa_ref[...], (((1,), (1,)), ((), ())),
                         preferred_element_type=jnp.float32)
    h = jnp.maximum(_bn(y2, g2_ref[...], b2_ref[...], 1), 0.0)
    y3 = lax.dot_general(w3_ref[...], h, (((1,), (0,)), ((), ())),
                         preferred_element_type=jnp.float32)
    z = _bn(y3, g3_ref[...], b3_ref[...], 1)
    o_ref[...] = jnp.maximum(z + f_ref[...], 0.0)


def _sc_body(f1t_hbm, idxf_hbm, pe_hbm, out_hbm,
             idx_v, fj_v, pe_v, out_v, sem_g, sem_p):
    nc = lax.axis_index("c")
    ns = lax.axis_index("s")
    wid = ns * 2 + nc
    base = jnp.minimum(wid * CHUNK, N - CHUNK)

    iota = lax.broadcasted_iota(jnp.int32, (16,), 0)

    def step(j, _):
        n0 = base + j * NB
        pltpu.sync_copy(idxf_hbm.at[pl.ds(n0 * K, NB * K)], idx_v)
        g = pltpu.async_copy(f1t_hbm.at[idx_v], fj_v, sem_g)
        p = pltpu.async_copy(pe_hbm.at[:, pl.ds(n0, NB), :], pe_v, sem_p)
        g.wait()
        p.wait()
        for jj in range(NB):
            jvec = jnp.full((16,), jj, jnp.int32)
            for cc in range(8):
                cvec = cc * 16 + iota

                def kbody(k, acc):
                    a = fj_v[jj * K + k, pl.ds(cc * 16, 16)]
                    b = plsc.load_gather(
                        pe_v, [cvec, jvec, jnp.full((16,), 0, jnp.int32) + k])
                    return jnp.maximum(acc, a + b)

                acc = lax.fori_loop(0, K, kbody,
                                    jnp.full((16,), -jnp.inf, jnp.float32))
                out_v[jj, pl.ds(cc * 16, 16)] = acc
        pltpu.sync_copy(out_v, out_hbm.at[pl.ds(n0, NB), :])
        return 0

    lax.fori_loop(0, STEPS, step, 0)


def _sc_stage(f1t, idx_flat, pe3):
    mesh = plsc.VectorSubcoreMesh(core_axis_name="c", subcore_axis_name="s")
    return pl.kernel(
        _sc_body,
        out_type=jax.ShapeDtypeStruct((N, C), jnp.float32),
        mesh=mesh,
        scratch_types=[
            pltpu.VMEM((NB * K,), jnp.int32),
            pltpu.VMEM((NB * K, C), jnp.float32),
            pltpu.VMEM((C, NB, K), jnp.float32),
            pltpu.VMEM((NB, C), jnp.float32),
            pltpu.SemaphoreType.DMA,
            pltpu.SemaphoreType.DMA,
        ],
    )(f1t, idx_flat, pe3)


def kernel(p, f, pe, idx, w1, g1, b1, w2, g2, b2, w3, g3, b3):
    f2d = f[0]
    pe3 = pe[0]
    idx_flat = idx[0].reshape(N * K)

    f1t = pl.pallas_call(
        _tc1_body,
        out_shape=jax.ShapeDtypeStruct((N, C), jnp.float32),
    )(f2d, w1, g1.reshape(1, C), b1.reshape(1, C))

    faggt = _sc_stage(f1t, idx_flat, pe3)

    out2d = pl.pallas_call(
        _tc2_body,
        out_shape=jax.ShapeDtypeStruct((C, N), jnp.float32),
    )(faggt, f2d, w2, g2.reshape(C, 1), b2.reshape(C, 1),
      w3, g3.reshape(C, 1), b3.reshape(C, 1))

    return (p, out2d[None], pe)


# speedup vs baseline: 2.1390x; 2.1390x over previous
import functools

import jax
import jax.numpy as jnp
from jax import lax
from jax.experimental import pallas as pl
from jax.experimental.pallas import tpu as pltpu
from jax.experimental.pallas import tpu_sc as plsc

C = 128
K = 32
N = 10000

NW = 32
CHUNK = 320
NB = 4
STEPS = CHUNK // NB


def _bn(y, g, b, axis):
    m = jnp.mean(y, axis=axis, keepdims=True)
    v = jnp.mean((y - m) ** 2, axis=axis, keepdims=True)
    return (y - m) * lax.rsqrt(v + 1e-5) * g + b


def _tc1_body(f_ref, w_ref, g_ref, b_ref, o_ref):
    y = lax.dot_general(f_ref[...], w_ref[...], (((0,), (1,)), ((), ())),
                        preferred_element_type=jnp.float32)
    o_ref[...] = jnp.maximum(_bn(y, g_ref[...], b_ref[...], 0), 0.0)


def _tc2_body(fa_ref, f_ref, w2_ref, g2_ref, b2_ref, w3_ref, g3_ref, b3_ref,
              o_ref):
    y2 = lax.dot_general(w2_ref[...], fa_ref[...], (((1,), (1,)), ((), ())),
                         preferred_element_type=jnp.float32)
    h = jnp.maximum(_bn(y2, g2_ref[...], b2_ref[...], 1), 0.0)
    y3 = lax.dot_general(w3_ref[...], h, (((1,), (0,)), ((), ())),
                         preferred_element_type=jnp.float32)
    z = _bn(y3, g3_ref[...], b3_ref[...], 1)
    o_ref[...] = jnp.maximum(z + f_ref[...], 0.0)


def _sc_body(f1t_hbm, idxf_hbm, pe_hbm, out_hbm,
             idx_v, fj_v, pe_v, out_v, sem_g, sem_p):
    nc = lax.axis_index("c")
    ns = lax.axis_index("s")
    wid = ns * 2 + nc
    base = jnp.minimum(wid * CHUNK, N - CHUNK)

    iota = lax.broadcasted_iota(jnp.int32, (16,), 0)

    def step(j, _):
        n0 = base + j * NB
        pltpu.sync_copy(idxf_hbm.at[pl.ds(n0 * K, NB * K)], idx_v)
        g = pltpu.async_copy(f1t_hbm.at[idx_v], fj_v, sem_g)
        p = pltpu.async_copy(pe_hbm.at[:, pl.ds(n0 * K, NB * K)], pe_v, sem_p)
        g.wait()
        p.wait()
        for jj in range(NB):
            for cc in range(8):
                cvec = cc * 16 + iota

                def kbody(k, acc):
                    a = fj_v[jj * K + k, pl.ds(cc * 16, 16)]
                    b = plsc.load_gather(
                        pe_v, [cvec, jnp.full((16,), jj * K, jnp.int32) + k])
                    return jnp.maximum(acc, a + b)

                acc = lax.fori_loop(0, K, kbody,
                                    jnp.full((16,), -jnp.inf, jnp.float32))
                out_v[jj, pl.ds(cc * 16, 16)] = acc
        pltpu.sync_copy(out_v, out_hbm.at[pl.ds(n0, NB), :])
        return 0

    lax.fori_loop(0, STEPS, step, 0)


def _sc_stage(f1t, idx_flat, pe3):
    mesh = plsc.VectorSubcoreMesh(core_axis_name="c", subcore_axis_name="s")
    return pl.kernel(
        _sc_body,
        out_type=jax.ShapeDtypeStruct((N, C), jnp.float32),
        mesh=mesh,
        compiler_params=pltpu.CompilerParams(needs_layout_passes=False),
        scratch_types=[
            pltpu.VMEM((NB * K,), jnp.int32),
            pltpu.VMEM((NB * K, C), jnp.float32),
            pltpu.VMEM((C, NB * K), jnp.float32),
            pltpu.VMEM((NB, C), jnp.float32),
            pltpu.SemaphoreType.DMA,
            pltpu.SemaphoreType.DMA,
        ],
    )(f1t, idx_flat, pe3)


def kernel(p, f, pe, idx, w1, g1, b1, w2, g2, b2, w3, g3, b3):
    f2d = f[0]
    pe2 = pe[0].reshape(C, N * K)
    idx_flat = idx[0].reshape(N * K)

    f1t = pl.pallas_call(
        _tc1_body,
        out_shape=jax.ShapeDtypeStruct((N, C), jnp.float32),
    )(f2d, w1, g1.reshape(1, C), b1.reshape(1, C))

    faggt = _sc_stage(f1t, idx_flat, pe2)

    out2d = pl.pallas_call(
        _tc2_body,
        out_shape=jax.ShapeDtypeStruct((C, N), jnp.float32),
    )(faggt, f2d, w2, g2.reshape(C, 1), b2.reshape(C, 1),
      w3, g3.reshape(C, 1), b3.reshape(C, 1))

    return (p, out2d[None], pe)


# speedup vs baseline: 3.3451x; 1.5638x over previous
import functools

import jax
import jax.numpy as jnp
from jax import lax
from jax.experimental import pallas as pl
from jax.experimental.pallas import tpu as pltpu
from jax.experimental.pallas import tpu_sc as plsc

C = 128
K = 32
N = 10000

NW = 32
CHUNK = 320
NB = 4
STEPS = CHUNK // NB


def _bn(y, g, b, axis):
    m = jnp.mean(y, axis=axis, keepdims=True)
    v = jnp.mean((y - m) ** 2, axis=axis, keepdims=True)
    return (y - m) * lax.rsqrt(v + 1e-5) * g + b


def _tc1_body(f_ref, w_ref, g_ref, b_ref, o_ref):
    y = lax.dot_general(f_ref[...], w_ref[...], (((0,), (1,)), ((), ())),
                        preferred_element_type=jnp.float32)
    o_ref[...] = jnp.maximum(_bn(y, g_ref[...], b_ref[...], 0), 0.0)


def _tc2_body(fa_ref, f_ref, w2_ref, g2_ref, b2_ref, w3_ref, g3_ref, b3_ref,
              o_ref):
    y2 = lax.dot_general(w2_ref[...], fa_ref[...], (((1,), (1,)), ((), ())),
                         preferred_element_type=jnp.float32)
    h = jnp.maximum(_bn(y2, g2_ref[...], b2_ref[...], 1), 0.0)
    y3 = lax.dot_general(w3_ref[...], h, (((1,), (0,)), ((), ())),
                         preferred_element_type=jnp.float32)
    z = _bn(y3, g3_ref[...], b3_ref[...], 1)
    o_ref[...] = jnp.maximum(z + f_ref[...], 0.0)


def _sc_body(f1t_hbm, idxf_hbm, pe_hbm, out_hbm,
             idx_v, fj_v, pe_v, out_v, sem_g, sem_p, sem_o):
    nc = lax.axis_index("c")
    ns = lax.axis_index("s")
    wid = ns * 2 + nc
    base = jnp.minimum(wid * CHUNK, N - CHUNK)

    iota = lax.broadcasted_iota(jnp.int32, (16,), 0)

    pltpu.sync_copy(idxf_hbm.at[pl.ds(base * K, CHUNK * K)], idx_v)

    def start(j, b):
        n0 = base + j * NB
        pltpu.async_copy(
            f1t_hbm.at[idx_v.at[pl.ds(j * (NB * K), NB * K)]],
            fj_v.at[b], sem_g.at[b])
        pltpu.async_copy(
            pe_hbm.at[:, pl.ds(n0 * K, NB * K)], pe_v.at[b], sem_p.at[b])

    def compute(j, b):
        n0 = base + j * NB
        pltpu.make_async_copy(
            f1t_hbm.at[idx_v.at[pl.ds(j * (NB * K), NB * K)]],
            fj_v.at[b], sem_g.at[b]).wait()
        pltpu.make_async_copy(
            pe_hbm.at[:, pl.ds(n0 * K, NB * K)], pe_v.at[b],
            sem_p.at[b]).wait()
        for jj in range(NB):
            for cc in range(8):
                cvec = cc * 16 + iota
                acc = None
                for k in range(K):
                    a = fj_v[b, jj * K + k, pl.ds(cc * 16, 16)]
                    pv = plsc.load_gather(
                        pe_v.at[b],
                        [cvec, jnp.full((16,), jj * K + k, jnp.int32)])
                    s = a + pv
                    acc = s if acc is None else jnp.maximum(acc, s)
                out_v[b, jj, pl.ds(cc * 16, 16)] = acc
        pltpu.async_copy(out_v.at[b], out_hbm.at[pl.ds(n0, NB), :], sem_o.at[b])

    def drain_out(j, b):
        n0 = base + j * NB
        pltpu.make_async_copy(
            out_v.at[b], out_hbm.at[pl.ds(n0, NB), :], sem_o.at[b]).wait()

    start(0, 0)

    def pair(j2, _):
        j = j2 * 2
        start(j + 1, 1)

        @pl.when(j2 >= 1)
        def _():
            drain_out(j - 2, 0)
        compute(j, 0)

        @pl.when(j2 + 1 < STEPS // 2)
        def _():
            start(j + 2, 0)

        @pl.when(j2 >= 1)
        def _():
            drain_out(j - 1, 1)
        compute(j + 1, 1)
        return 0

    lax.fori_loop(0, STEPS // 2, pair, 0)
    drain_out(STEPS - 2, 0)
    drain_out(STEPS - 1, 1)


def _sc_stage(f1t, idx_flat, pe3):
    mesh = plsc.VectorSubcoreMesh(core_axis_name="c", subcore_axis_name="s")
    return pl.kernel(
        _sc_body,
        out_type=jax.ShapeDtypeStruct((N, C), jnp.float32),
        mesh=mesh,
        compiler_params=pltpu.CompilerParams(needs_layout_passes=False),
        scratch_types=[
            pltpu.VMEM((CHUNK * K,), jnp.int32),
            pltpu.VMEM((2, NB * K, C), jnp.float32),
            pltpu.VMEM((2, C, NB * K), jnp.float32),
            pltpu.VMEM((2, NB, C), jnp.float32),
            pltpu.SemaphoreType.DMA((2,)),
            pltpu.SemaphoreType.DMA((2,)),
            pltpu.SemaphoreType.DMA((2,)),
        ],
    )(f1t, idx_flat, pe3)


def kernel(p, f, pe, idx, w1, g1, b1, w2, g2, b2, w3, g3, b3):
    f2d = f[0]
    pe2 = pe[0].reshape(C, N * K)
    idx_flat = idx[0].reshape(N * K)

    f1t = pl.pallas_call(
        _tc1_body,
        out_shape=jax.ShapeDtypeStruct((N, C), jnp.float32),
    )(f2d, w1, g1.reshape(1, C), b1.reshape(1, C))

    faggt = _sc_stage(f1t, idx_flat, pe2)

    out2d = pl.pallas_call(
        _tc2_body,
        out_shape=jax.ShapeDtypeStruct((C, N), jnp.float32),
    )(faggt, f2d, w2, g2.reshape(C, 1), b2.reshape(C, 1),
      w3, g3.reshape(C, 1), b3.reshape(C, 1))

    return (p, out2d[None], pe)
